# baseline (speedup 1.0000x reference)
import functools

import jax
import jax.numpy as jnp
from jax.experimental import pallas as pl
from jax.experimental.pallas import tpu as pltpu
from jax.experimental.pallas import tpu_sc as plsc

VOCAB = 100000
EMBED = 64
CTX = 3
HIDDEN = 128
BATCH = 1024

V_BLK = 2048
NV = (VOCAB + V_BLK - 1) // V_BLK

GATHER_WINDOW = 128


def _sc_gather(table128, flat_idx):
    n_idx = flat_idx.shape[1]
    mesh = plsc.VectorSubcoreMesh(core_axis_name="core", subcore_axis_name="subcore")

    @pl.kernel(
        out_type=jax.ShapeDtypeStruct((n_idx, 128), table128.dtype),
        mesh=mesh,
    )
    def gather_kernel(tbl_hbm, idx_hbm, out_hbm):
        def body(i_vmem, o_vmem):
            pltpu.sync_copy(tbl_hbm.at[i_vmem.at[0]], o_vmem)

        pltpu.emit_pipeline(
            body,
            grid=(n_idx // GATHER_WINDOW,),
            in_specs=[
                pl.BlockSpec((1, GATHER_WINDOW), index_map=lambda i: (0, i))
            ],
            out_specs=[
                pl.BlockSpec((GATHER_WINDOW, 128), index_map=lambda i: (i, 0))
            ],
            core_axis_name="subcore",
            dimension_semantics=(pltpu.PARALLEL,),
        )(idx_hbm, out_hbm)

    return gather_kernel(table128, flat_idx)


def _pass_a_kernel(embeds_ref, w1_ref, b1_ref, w2_ref, b2_ref,
                   hid_out, lse_out, hid_scr, m_scr, s_scr):
    j = pl.program_id(0)

    @pl.when(j == 0)
    def _init():
        h = jnp.tanh(
            jnp.dot(embeds_ref[...], w1_ref[...],
                    preferred_element_type=jnp.float32)
            + b1_ref[...]
        )
        hb = h.astype(jnp.bfloat16)
        hid_scr[...] = hb
        hid_out[...] = hb
        m_scr[...] = jnp.full((BATCH, 1), -jnp.inf, dtype=jnp.float32)
        s_scr[...] = jnp.zeros((BATCH, 1), dtype=jnp.float32)

    logits = (
        jnp.dot(hid_scr[...], w2_ref[...], preferred_element_type=jnp.float32)
        + b2_ref[...]
    )
    cols = j * V_BLK + jax.lax.broadcasted_iota(jnp.int32, (1, V_BLK), 1)
    logits = jnp.where(cols < VOCAB, logits, -jnp.inf)

    m_old = m_scr[...]
    block_max = jnp.max(logits, axis=1, keepdims=True)
    m_new = jnp.maximum(m_old, block_max)
    block_sum = jnp.sum(jnp.exp(logits - m_new), axis=1, keepdims=True)
    s_scr[...] = s_scr[...] * jnp.exp(m_old - m_new) + block_sum
    m_scr[...] = m_new

    @pl.when(j == NV - 1)
    def _finish():
        lse_out[...] = m_scr[...] + jnp.log(s_scr[...])


def _pass_b_kernel(hid_ref, w2_ref, b2_ref, lse_ref, out_ref):
    logits = (
        jnp.dot(hid_ref[...], w2_ref[...], preferred_element_type=jnp.float32)
        + b2_ref[...]
    )
    out_ref[...] = logits - lse_ref[...]


def kernel(context_words, table, W1, b1, W2, b2):
    flat_idx = context_words.reshape(1, BATCH * CTX)
    table128 = jnp.pad(table, ((0, 0), (0, 128 - EMBED)))
    embeds = _sc_gather(table128, flat_idx)
    embeds = embeds.reshape(BATCH, CTX * 128)

    w1_pad = jnp.pad(
        W1.reshape(CTX, EMBED, HIDDEN), ((0, 0), (0, 128 - EMBED), (0, 0))
    ).reshape(CTX * 128, HIDDEN)

    w2_bf16 = W2.astype(jnp.bfloat16)
    b1r = b1.reshape(1, HIDDEN)
    b2r = b2.reshape(1, VOCAB)

    hid_bf16, lse = pl.pallas_call(
        _pass_a_kernel,
        grid=(NV,),
        in_specs=[
            pl.BlockSpec((BATCH, CTX * 128), lambda j: (0, 0)),
            pl.BlockSpec((CTX * 128, HIDDEN), lambda j: (0, 0)),
            pl.BlockSpec((1, HIDDEN), lambda j: (0, 0)),
            pl.BlockSpec((HIDDEN, V_BLK), lambda j: (0, j)),
            pl.BlockSpec((1, V_BLK), lambda j: (0, j)),
        ],
        out_specs=[
            pl.BlockSpec((BATCH, HIDDEN), lambda j: (0, 0)),
            pl.BlockSpec((BATCH, 1), lambda j: (0, 0)),
        ],
        out_shape=[
            jax.ShapeDtypeStruct((BATCH, HIDDEN), jnp.bfloat16),
            jax.ShapeDtypeStruct((BATCH, 1), jnp.float32),
        ],
        scratch_shapes=[
            pltpu.VMEM((BATCH, HIDDEN), jnp.bfloat16),
            pltpu.VMEM((BATCH, 1), jnp.float32),
            pltpu.VMEM((BATCH, 1), jnp.float32),
        ],
    )(embeds, w1_pad, b1r, w2_bf16, b2r)

    out = pl.pallas_call(
        _pass_b_kernel,
        grid=(NV,),
        in_specs=[
            pl.BlockSpec((BATCH, HIDDEN), lambda j: (0, 0)),
            pl.BlockSpec((HIDDEN, V_BLK), lambda j: (0, j)),
            pl.BlockSpec((1, V_BLK), lambda j: (0, j)),
            pl.BlockSpec((BATCH, 1), lambda j: (0, 0)),
        ],
        out_specs=pl.BlockSpec((BATCH, V_BLK), lambda j: (0, j)),
        out_shape=jax.ShapeDtypeStruct((BATCH, VOCAB), jnp.float32),
    )(hid_bf16, w2_bf16, b2r, lse)

    return out


# speedup vs baseline: 1.0103x; 1.0103x over previous
import functools

import jax
import jax.numpy as jnp
from jax.experimental import pallas as pl
from jax.experimental.pallas import tpu as pltpu
from jax.experimental.pallas import tpu_sc as plsc

VOCAB = 100000
EMBED = 64
CTX = 3
HIDDEN = 128
BATCH = 1024

V_BLK = 2048
NV = (VOCAB + V_BLK - 1) // V_BLK

GATHER_WINDOW = 128


def _sc_gather(table128, flat_idx):
    n_idx = flat_idx.shape[1]
    mesh = plsc.VectorSubcoreMesh(core_axis_name="core", subcore_axis_name="subcore")

    @pl.kernel(
        out_type=jax.ShapeDtypeStruct((n_idx, 128), table128.dtype),
        mesh=mesh,
    )
    def gather_kernel(tbl_hbm, idx_hbm, out_hbm):
        def body(i_vmem, o_vmem):
            pltpu.sync_copy(tbl_hbm.at[i_vmem.at[0]], o_vmem)

        pltpu.emit_pipeline(
            body,
            grid=(n_idx // GATHER_WINDOW,),
            in_specs=[
                pl.BlockSpec((1, GATHER_WINDOW), index_map=lambda i: (0, i))
            ],
            out_specs=[
                pl.BlockSpec((GATHER_WINDOW, 128), index_map=lambda i: (i, 0))
            ],
            core_axis_name="subcore",
            dimension_semantics=(pltpu.PARALLEL,),
        )(idx_hbm, out_hbm)

    return gather_kernel(table128, flat_idx)


def _pass_a_kernel(embeds_ref, w1_ref, b1_ref, w2_ref, b2_ref,
                   hid_out, lse_out, tail_out, hid_scr, m_scr, s_scr):
    j = pl.program_id(0)

    @pl.when(j == 0)
    def _init():
        h = jnp.tanh(
            jnp.dot(embeds_ref[...], w1_ref[...],
                    preferred_element_type=jnp.float32)
            + b1_ref[...]
        )
        hb = h.astype(jnp.bfloat16)
        hid_scr[...] = hb
        hid_out[...] = hb
        m_scr[...] = jnp.full((BATCH, 1), -jnp.inf, dtype=jnp.float32)
        s_scr[...] = jnp.zeros((BATCH, 1), dtype=jnp.float32)

    logits = (
        jnp.dot(hid_scr[...], w2_ref[...], preferred_element_type=jnp.float32)
        + b2_ref[...]
    )
    cols = j * V_BLK + jax.lax.broadcasted_iota(jnp.int32, (1, V_BLK), 1)
    logits = jnp.where(cols < VOCAB, logits, -jnp.inf)

    m_old = m_scr[...]
    block_max = jnp.max(logits, axis=1, keepdims=True)
    m_new = jnp.maximum(m_old, block_max)
    block_sum = jnp.sum(jnp.exp(logits - m_new), axis=1, keepdims=True)
    s_scr[...] = s_scr[...] * jnp.exp(m_old - m_new) + block_sum
    m_scr[...] = m_new

    @pl.when(j == NV - 1)
    def _finish():
        lse = m_scr[...] + jnp.log(s_scr[...])
        lse_out[...] = lse
        tail_out[...] = logits - lse


NBUF = 3
NCHUNK = 8
CHUNK = V_BLK // NCHUNK
NV_FULL = NV - 1


def _b_chunk_copy(scr, sems, out_ref, step, buf):
    return [
        pltpu.make_async_copy(
            scr.at[buf, :, pl.ds(c * CHUNK, CHUNK)],
            out_ref.at[:, pl.ds(step * V_BLK + c * CHUNK, CHUNK)],
            sems.at[buf, c],
        )
        for c in range(NCHUNK)
    ]


def _pass_b_kernel(hid_ref, w2_ref, b2_ref, lse_ref, tail_in, out_ref,
                   scr, sems):
    del tail_in
    j = pl.program_id(0)
    buf = jax.lax.rem(j, NBUF)

    @pl.when(j >= NBUF)
    def _reclaim():
        for cp in _b_chunk_copy(scr, sems, out_ref, j - NBUF, buf):
            cp.wait()

    logits = (
        jnp.dot(hid_ref[...], w2_ref[...], preferred_element_type=jnp.float32)
        + b2_ref[...]
    )
    scr[buf] = logits - lse_ref[...]

    for cp in _b_chunk_copy(scr, sems, out_ref, j, buf):
        cp.start()

    @pl.when(j == NV_FULL - 1)
    def _drain():
        for d in range(NBUF - 1, -1, -1):
            s = NV_FULL - 1 - d
            for cp in _b_chunk_copy(scr, sems, out_ref, s, s % NBUF):
                cp.wait()


def kernel(context_words, table, W1, b1, W2, b2):
    flat_idx = context_words.reshape(1, BATCH * CTX)
    table128 = jnp.pad(table, ((0, 0), (0, 128 - EMBED)))
    embeds = _sc_gather(table128, flat_idx)
    embeds = embeds.reshape(BATCH, CTX * 128)

    w1_pad = jnp.pad(
        W1.reshape(CTX, EMBED, HIDDEN), ((0, 0), (0, 128 - EMBED), (0, 0))
    ).reshape(CTX * 128, HIDDEN)

    w2_bf16 = W2.astype(jnp.bfloat16)
    b1r = b1.reshape(1, HIDDEN)
    b2r = b2.reshape(1, VOCAB)

    hid_bf16, lse, out_with_tail = pl.pallas_call(
        _pass_a_kernel,
        grid=(NV,),
        in_specs=[
            pl.BlockSpec((BATCH, CTX * 128), lambda j: (0, 0)),
            pl.BlockSpec((CTX * 128, HIDDEN), lambda j: (0, 0)),
            pl.BlockSpec((1, HIDDEN), lambda j: (0, 0)),
            pl.BlockSpec((HIDDEN, V_BLK), lambda j: (0, j)),
            pl.BlockSpec((1, V_BLK), lambda j: (0, j)),
        ],
        out_specs=[
            pl.BlockSpec((BATCH, HIDDEN), lambda j: (0, 0)),
            pl.BlockSpec((BATCH, 1), lambda j: (0, 0)),
            pl.BlockSpec((BATCH, V_BLK), lambda j: (0, NV - 1)),
        ],
        out_shape=[
            jax.ShapeDtypeStruct((BATCH, HIDDEN), jnp.bfloat16),
            jax.ShapeDtypeStruct((BATCH, 1), jnp.float32),
            jax.ShapeDtypeStruct((BATCH, VOCAB), jnp.float32),
        ],
        scratch_shapes=[
            pltpu.VMEM((BATCH, HIDDEN), jnp.bfloat16),
            pltpu.VMEM((BATCH, 1), jnp.float32),
            pltpu.VMEM((BATCH, 1), jnp.float32),
        ],
    )(embeds, w1_pad, b1r, w2_bf16, b2r)

    out = pl.pallas_call(
        _pass_b_kernel,
        grid=(NV_FULL,),
        in_specs=[
            pl.BlockSpec((BATCH, HIDDEN), lambda j: (0, 0)),
            pl.BlockSpec((HIDDEN, V_BLK), lambda j: (0, j)),
            pl.BlockSpec((1, V_BLK), lambda j: (0, j)),
            pl.BlockSpec((BATCH, 1), lambda j: (0, 0)),
            pl.BlockSpec(memory_space=pltpu.MemorySpace.HBM),
        ],
        out_specs=pl.BlockSpec(memory_space=pltpu.MemorySpace.HBM),
        out_shape=jax.ShapeDtypeStruct((BATCH, VOCAB), jnp.float32),
        scratch_shapes=[
            pltpu.VMEM((NBUF, BATCH, V_BLK), jnp.float32),
            pltpu.SemaphoreType.DMA((NBUF, NCHUNK)),
        ],
        input_output_aliases={4: 0},
    )(hid_bf16, w2_bf16, b2r, lse, out_with_tail)

    return out
